# baseline (speedup 1.0000x reference)
import functools

import jax
import jax.numpy as jnp
from jax import lax
from jax.experimental import pallas as pl
from jax.experimental.pallas import tpu as pltpu
from jax.experimental.pallas import tpu_sc as plsc

EMBED_DIM = 64
SCALE = float(EMBED_DIM) ** 0.5

NUM_CORES = 2
NUM_SUBCORES = 16
NUM_WORKERS = NUM_CORES * NUM_SUBCORES
CHUNK = 100
GSIZE = 2
NSETS = 3


def _make_lookup(batch: int, hist: int):
    n_total = batch * hist
    per_worker = n_total // NUM_WORKERS
    n_chunks = per_worker // CHUNK
    n_groups = n_chunks // GSIZE
    assert n_chunks == n_groups * GSIZE
    n_main = (n_groups - 5) // NSETS
    assert 2 + NSETS * n_main == n_groups - 3

    mesh = plsc.VectorSubcoreMesh(
        core_axis_name="c", subcore_axis_name="s",
        num_cores=NUM_CORES, num_subcores=NUM_SUBCORES)

    @functools.partial(
        pl.kernel,
        out_type=jax.ShapeDtypeStruct((batch, hist, EMBED_DIM), jnp.float32),
        mesh=mesh,
        scratch_types=[
            pltpu.VMEM((n_chunks, CHUNK), jnp.int32),
            [pltpu.VMEM((GSIZE, CHUNK, EMBED_DIM), jnp.float32)
             for _ in range(NSETS)],
            [pltpu.SemaphoreType.DMA for _ in range(NSETS)],
            [pltpu.SemaphoreType.DMA for _ in range(NSETS)],
        ],
        compiler_params=pltpu.CompilerParams(use_tc_tiling_on_sc=False),
    )
    def lookup(x_hbm, table_hbm, out_hbm, idx_v, bufs, gsems, ssems):
        wid = lax.axis_index("s") * NUM_CORES + lax.axis_index("c")
        kbase = wid * n_chunks
        pltpu.sync_copy(x_hbm.at[pl.ds(kbase, n_chunks)], idx_v)

        def out_slice(g, b):
            k = kbase + g * GSIZE + b
            row = lax.shift_right_logical(k, 1)
            h0 = lax.mul(lax.bitwise_and(k, 1), CHUNK)
            return out_hbm.at[row, pl.ds(h0, CHUNK), :]

        def start_gathers(s, g):
            for b in range(GSIZE):
                pltpu.async_copy(
                    table_hbm.at[idx_v.at[g * GSIZE + b]], bufs[s].at[b],
                    gsems[s])

        def wait_gathers(s):
            for b in range(GSIZE):
                pltpu.make_async_copy(
                    table_hbm.at[idx_v.at[b]], bufs[s].at[b],
                    gsems[s]).wait()

        def scale_and_store(s, g):
            buf = bufs[s]
            for b in range(GSIZE):
                @plsc.parallel_loop(0, CHUNK, unroll=4)
                def _(r):
                    for k in range(EMBED_DIM // 16):
                        sl = pl.ds(k * 16, 16)
                        buf[b, r, sl] = buf[b, r, sl] * SCALE
                pltpu.async_copy(buf.at[b], out_slice(g, b), ssems[s])

        def wait_stores(s):
            for b in range(GSIZE):
                pltpu.make_async_copy(
                    bufs[s].at[b], out_slice(0, b), ssems[s]).wait()

        start_gathers(0, 0)
        start_gathers(1, 1)
        wait_gathers(0)
        scale_and_store(0, 0)
        start_gathers(2, 2)
        wait_gathers(1)
        scale_and_store(1, 1)

        def main_body(t, _):
            g0 = NSETS * t + 2
            for i, s in enumerate((2, 0, 1)):
                g = g0 + i
                wait_stores((s + 1) % NSETS)
                start_gathers((s + 1) % NSETS, g + 1)
                wait_gathers(s)
                scale_and_store(s, g)
            return ()

        lax.fori_loop(0, n_main, main_body, ())

        gT = n_groups - 3
        wait_stores(0)
        start_gathers(0, gT + 1)
        wait_gathers(2)
        scale_and_store(2, gT)

        wait_stores(1)
        start_gathers(1, gT + 2)
        wait_gathers(0)
        scale_and_store(0, gT + 1)

        wait_stores(2)
        wait_gathers(1)
        scale_and_store(1, gT + 2)

        wait_stores(0)
        wait_stores(1)

    return lookup


def kernel(x, table):
    batch, hist = x.shape
    x_flat = x.reshape(batch * hist // CHUNK, CHUNK).astype(jnp.int32)
    return _make_lookup(batch, hist)(x_flat, table)
